# baseline (speedup 1.0000x reference)
import functools

import jax
import jax.numpy as jnp
from jax import lax
from jax.experimental import pallas as pl
from jax.experimental.pallas import tpu as pltpu
from jax.experimental.pallas import tpu_sc as plsc

B = 16384
HID = 32
OUT = 3
NBIN = 11
VSPC, VNTA = 133, 188
OFF_SPC = 1 << NBIN
VSPC4 = 136
OFF_NTA = OFF_SPC + VSPC4
VTOT = OFF_NTA + VNTA
DBIG = 16
TOT = 2 * NBIN + 2 * DBIG + 1
NC, NS, L = 2, 16, 16
NW = NC * NS
BPW = B // NW
GR = 128
NG = BPW // GR
PK = 4
B4 = B // PK
PPW = BPW // PK

_dn = (((1,), (1,)), ((), ()))


def _prep_tables(bin_embs, emb_spc, emb_nta, W1, b1, W2, b2, W3, b3):
    def body(*refs):
        eb = refs[:NBIN]
        (espc_ref, enta_ref, w1_ref, b1_ref, w2_ref, b2_ref, w3_ref, b3_ref,
         tab_ref, w2p_ref, w3p_ref, mn_ref, b2p_ref, b3p_ref) = refs[NBIN:]
        w1 = w1_ref[...]
        const = b1_ref[...][None, :]
        deltas = []
        for c in range(NBIN):
            tc = lax.dot_general(eb[c][...], w1[:, 2 * c:2 * c + 2], _dn,
                                 preferred_element_type=jnp.float32)
            const = const + tc[0:1]
            deltas.append(tc[1:2] - tc[0:1])
        delta = jnp.concatenate(deltas, axis=0)

        jj = lax.broadcasted_iota(jnp.int32, ((1 << NBIN) // PK, PK * NBIN), 0)
        qq = lax.broadcasted_iota(jnp.int32, ((1 << NBIN) // PK, PK * NBIN), 1)
        bitsp = (((PK * jj + qq // NBIN) >> (qq % NBIN)) & 1).astype(jnp.float32)
        zc = jnp.zeros((1, HID), jnp.float32)
        zd = jnp.zeros((NBIN, HID), jnp.float32)
        deltap = jnp.concatenate(
            [jnp.concatenate([delta if i == k else zd for k in range(PK)],
                             axis=1) for i in range(PK)], axis=0)
        constp = jnp.concatenate([const] * PK, axis=1)
        bin_p = constp + jnp.dot(bitsp, deltap,
                                 preferred_element_type=jnp.float32)

        off = 2 * NBIN
        t_spc = lax.dot_general(espc_ref[...], w1[:, off:off + DBIG], _dn,
                                preferred_element_type=jnp.float32)
        t_nta = lax.dot_general(enta_ref[...], w1[:, off + DBIG:off + 2 * DBIG],
                                _dn, preferred_element_type=jnp.float32)

        def pack_rows(t, vp):
            v = t.shape[0]
            cols = []
            for a in range(PK):
                ji = lax.broadcasted_iota(jnp.int32, (vp // PK, v), 0)
                ri = lax.broadcasted_iota(jnp.int32, (vp // PK, v), 1)
                sel = (ri == PK * ji + a).astype(jnp.float32)
                cols.append(jnp.dot(sel, t, preferred_element_type=jnp.float32))
            return jnp.concatenate(cols, axis=1)

        spc_p = pack_rows(t_spc, VSPC4)
        nta_p = pack_rows(t_nta, VNTA)
        wnum = w1[:, TOT - 1:TOT]
        wnum_row = lax.dot_general(jnp.ones((1, 1), jnp.float32), wnum, _dn,
                                   preferred_element_type=jnp.float32)
        wnum_p = jnp.concatenate([wnum_row] * PK, axis=1)
        tab_ref[...] = jnp.concatenate([bin_p, spc_p, nta_p, wnum_p], axis=0)
        zc = jnp.zeros((1, HID), jnp.float32)
        mn_ref[...] = jnp.concatenate(
            [jnp.concatenate([wnum_row if i == k else zc for k in range(PK)],
                             axis=1) for i in range(PK)], axis=0)

        w2 = w2_ref[...]
        z32 = jnp.zeros((HID, HID), jnp.float32)
        w2p_ref[...] = jnp.concatenate(
            [jnp.concatenate([w2 if i == k else z32 for k in range(PK)], axis=1)
             for i in range(PK)], axis=0)
        w3 = w3_ref[...]
        z1 = jnp.zeros((1, HID), jnp.float32)
        w3p_ref[...] = jnp.concatenate(
            [jnp.concatenate([w3[o:o + 1] if k == a else z1
                              for k in range(PK)], axis=1)
             for o in range(OUT) for a in range(PK)], axis=0)
        b2r = b2_ref[...][None, :]
        b2p_ref[...] = jnp.concatenate([b2r] * PK, axis=1)
        b3r = b3_ref[...][None, :]
        b3p_ref[...] = jnp.concatenate(
            [b3r[:, o:o + 1] for o in range(OUT) for a in range(PK)],
            axis=1)

    out_shapes = (
        jax.ShapeDtypeStruct((VTOT // PK + 1, PK * HID), jnp.float32),
        jax.ShapeDtypeStruct((PK * HID, PK * HID), jnp.float32),
        jax.ShapeDtypeStruct((PK * OUT, PK * HID), jnp.float32),
        jax.ShapeDtypeStruct((PK, PK * HID), jnp.float32),
        jax.ShapeDtypeStruct((1, PK * HID), jnp.float32),
        jax.ShapeDtypeStruct((1, PK * OUT), jnp.float32),
    )
    return pl.pallas_call(body, out_shape=out_shapes)(
        *bin_embs, emb_spc, emb_nta, W1, b1, W2, b2, W3, b3)


def _sc_gather_sum(idxs_and_table):
    mesh = plsc.VectorSubcoreMesh(core_axis_name="c", subcore_axis_name="s")

    scratch = [pltpu.VMEM((BPW,), jnp.int32) for _ in range(NBIN + 2)]
    scratch += [pltpu.VMEM((BPW,), jnp.int32) for _ in range(3)]
    scratch += [pltpu.VMEM((BPW, HID), jnp.float32) for _ in range(3)]
    scratch.append(pltpu.VMEM((PPW, PK * HID), jnp.float32))
    scratch += [pltpu.SemaphoreType.DMA for _ in range(NG)]
    scratch += [pltpu.SemaphoreType.DMA for _ in range(NG)]
    scratch.append(pltpu.SemaphoreType.DMA)

    @functools.partial(
        pl.kernel,
        out_type=jax.ShapeDtypeStruct((B4, PK * HID), jnp.float32),
        mesh=mesh,
        scratch_types=scratch,
        compiler_params=pltpu.CompilerParams(use_tc_tiling_on_sc=False),
    )
    def body(*refs):
        idx_hbm = refs[:NBIN + 2]
        tab_hbm = refs[NBIN + 2]
        out = refs[NBIN + 3]
        idx_v = refs[NBIN + 4:2 * NBIN + 6]
        key_v = refs[2 * NBIN + 6:2 * NBIN + 9]
        r = refs[2 * NBIN + 9:2 * NBIN + 12]
        rp = refs[2 * NBIN + 12]
        isems = refs[2 * NBIN + 13:2 * NBIN + 13 + NG]
        gsems = refs[2 * NBIN + 13 + NG:2 * NBIN + 13 + 2 * NG]
        osem = refs[2 * NBIN + 13 + 2 * NG]

        wid = lax.axis_index("s") * NC + lax.axis_index("c")
        order = [NBIN, NBIN + 1] + list(range(NBIN))
        idx_cps = {}
        for c in order:
            idx_cps[c] = [
                pltpu.async_copy(idx_hbm[c].at[pl.ds(B4 * a + GR * wid, GR)],
                                 idx_v[c].at[pl.ds(a * GR, GR)], isems[a])
                for a in range(NG)]
        copies = [[None] * 3 for _ in range(NG)]
        for g in range(NG):
            for c in order:
                idx_cps[c][g].wait()

            def keys(k, _):
                lanes = pl.ds(k * L, L)
                key_v[1][lanes] = idx_v[NBIN][lanes] + OFF_SPC
                key_v[2][lanes] = idx_v[NBIN + 1][lanes] + OFF_NTA
                acc = idx_v[0][lanes]
                for c in range(1, NBIN):
                    acc = acc | (idx_v[c][lanes] << c)
                key_v[0][lanes] = acc
                return 0
            lax.fori_loop(g * (GR // L), (g + 1) * (GR // L), keys, 0,
                          unroll=2)
            rows = pl.ds(g * GR, GR)
            for t in range(3):
                copies[g][t] = pltpu.async_copy(
                    tab_hbm.at[key_v[t].at[rows]], r[t].at[rows], gsems[g])

        for g in range(NG):
            for t in range(3):
                copies[g][t].wait()

            @plsc.parallel_loop(0, PPW, 1, unroll=2)
            def _pack(pj, g=g):
                i = g * GR + pj
                for half in range(HID // L):
                    s = pl.ds(half * L, L)
                    rp[pj, pl.ds(g * HID + half * L, L)] = (
                        r[0][i, s] + r[1][i, s] + r[2][i, s])

        pltpu.async_copy(rp, out.at[pl.ds(wid * PPW, PPW)], osem).wait()

    return body(*idxs_and_table)


def _tc_mlp(h4, n128, w2p, b2p, w3p, b3p, mn):
    BR4 = B4 // 2
    NB = BR4 // 128

    def body(h_ref, n0_ref, n1_ref, n2_ref, n3_ref, w2_ref, b2_ref,
             w3_ref, b3_ref, mn_ref, o_ref):
        m1a = lax.broadcasted_iota(jnp.int32, (BR4, NB), 0) // 128
        m1b = lax.broadcasted_iota(jnp.int32, (BR4, NB), 1)
        m1 = (m1a == m1b).astype(jnp.float32)
        da = lax.broadcasted_iota(jnp.int32, (BR4, 128), 0) % 128
        db = lax.broadcasted_iota(jnp.int32, (BR4, 128), 1)
        dmask = (da == db).astype(jnp.float32)
        cols = []
        for n_ref in (n0_ref, n1_ref, n2_ref, n3_ref):
            spread = jnp.dot(m1, n_ref[...],
                             preferred_element_type=jnp.float32)
            cols.append(jnp.sum(spread * dmask, axis=1, keepdims=True))
        numt4 = jnp.concatenate(cols, axis=1)

        h = h_ref[...] + jnp.dot(numt4, mn_ref[...],
                                 preferred_element_type=jnp.float32)
        h = jnp.maximum(h, 0.0)
        h = lax.dot_general(h, w2_ref[...], _dn,
                            preferred_element_type=jnp.float32) + b2_ref[...]
        h = jnp.maximum(h, 0.0)
        lo = lax.dot_general(h, w3_ref[...], _dn,
                             preferred_element_type=jnp.float32) + b3_ref[...]
        m = jnp.max(lo, axis=1, keepdims=True)
        e = jnp.exp(lo - m)
        qa = lax.broadcasted_iota(jnp.int32, (PK * OUT, PK * OUT), 0) % PK
        qb = lax.broadcasted_iota(jnp.int32, (PK * OUT, PK * OUT), 1) % PK
        q = (qa == qb).astype(jnp.float32)
        den = jnp.dot(e, q, preferred_element_type=jnp.float32)
        o_ref[...] = jnp.transpose(e / den)

    rep = lambda shape: pl.BlockSpec(shape, lambda i: tuple(0 for _ in shape))
    nspec = lambda a: pl.BlockSpec((NB, 128), lambda i, a=a: (2 * a + i, 0))
    return pl.pallas_call(
        body,
        grid=(B4 // BR4,),
        in_specs=[
            pl.BlockSpec((BR4, PK * HID), lambda i: (i, 0)),
            nspec(0), nspec(1), nspec(2), nspec(3),
            rep((PK * HID, PK * HID)),
            rep((1, PK * HID)),
            rep((PK * OUT, PK * HID)),
            rep((1, PK * OUT)),
            rep((PK, PK * HID)),
        ],
        out_specs=pl.BlockSpec((PK * OUT, BR4), lambda i: (0, i)),
        out_shape=jax.ShapeDtypeStruct((PK * OUT, B4), jnp.float32),
    )(h4, n128, n128, n128, n128, w2p, b2p, w3p, b3p, mn)


def kernel(numerical_features,
           idx_root_stone, emb_root_stone,
           idx_root_grate, emb_root_grate,
           idx_root_other, emb_root_other,
           idx_trunk_wire, emb_trunk_wire,
           idx_trnk_light, emb_trnk_light,
           idx_trnk_other, emb_trnk_other,
           idx_brch_light, emb_brch_light,
           idx_brch_shoe, emb_brch_shoe,
           idx_brch_other, emb_brch_other,
           idx_curb_loc, emb_curb_loc,
           idx_sidewalk, emb_sidewalk,
           idx_spc_common, emb_spc_common,
           idx_nta, emb_nta,
           W1, b1, W2, b2, W3, b3):
    idxs = [idx_root_stone, idx_root_grate, idx_root_other, idx_trunk_wire,
            idx_trnk_light, idx_trnk_other, idx_brch_light, idx_brch_shoe,
            idx_brch_other, idx_curb_loc, idx_sidewalk, idx_spc_common, idx_nta]
    bin_embs = [emb_root_stone, emb_root_grate, emb_root_other, emb_trunk_wire,
                emb_trnk_light, emb_trnk_other, emb_brch_light, emb_brch_shoe,
                emb_brch_other, emb_curb_loc, emb_sidewalk]

    tab4, w2p, w3p, mn, b2p, b3p = _prep_tables(
        bin_embs, emb_spc_common, emb_nta, W1, b1, W2, b2, W3, b3)
    tab = jnp.reshape(tab4, (VTOT + PK, HID))
    idxs32 = [i.astype(jnp.int32) for i in idxs]
    h4 = _sc_gather_sum(idxs32 + [tab])
    n128 = jnp.reshape(numerical_features, (128, 128))
    o12 = _tc_mlp(h4, n128, w2p, b2p, w3p, b3p, mn)
    return jnp.transpose(jnp.reshape(o12, (OUT, B)))


# speedup vs baseline: 1.0058x; 1.0058x over previous
import functools

import jax
import jax.numpy as jnp
from jax import lax
from jax.experimental import pallas as pl
from jax.experimental.pallas import tpu as pltpu
from jax.experimental.pallas import tpu_sc as plsc

B = 16384
HID = 32
OUT = 3
NBIN = 11
VSPC, VNTA = 133, 188
OFF_SPC = 1 << NBIN
VSPC4 = 136
OFF_NTA = OFF_SPC + VSPC4
VTOT = OFF_NTA + VNTA
DBIG = 16
TOT = 2 * NBIN + 2 * DBIG + 1
NC, NS, L = 2, 16, 16
NW = NC * NS
BPW = B // NW
GR = 128
NG = BPW // GR
PK = 4
B4 = B // PK
PPW = BPW // PK

_dn = (((1,), (1,)), ((), ()))


def _prep_tables(bin_embs, emb_spc, emb_nta, W1, b1, W2, b2, W3, b3):
    def body(*refs):
        eb = refs[:NBIN]
        (espc_ref, enta_ref, w1_ref, b1_ref, w2_ref, b2_ref, w3_ref, b3_ref,
         tab_ref, w2p_ref, w3p_ref, mn_ref, b2p_ref, b3p_ref) = refs[NBIN:]
        w1 = w1_ref[...]
        const = b1_ref[...][None, :]
        deltas = []
        for c in range(NBIN):
            tc = lax.dot_general(eb[c][...], w1[:, 2 * c:2 * c + 2], _dn,
                                 preferred_element_type=jnp.float32)
            const = const + tc[0:1]
            deltas.append(tc[1:2] - tc[0:1])
        delta = jnp.concatenate(deltas, axis=0)

        jj = lax.broadcasted_iota(jnp.int32, ((1 << NBIN) // PK, PK * NBIN), 0)
        qq = lax.broadcasted_iota(jnp.int32, ((1 << NBIN) // PK, PK * NBIN), 1)
        bitsp = (((PK * jj + qq // NBIN) >> (qq % NBIN)) & 1).astype(jnp.float32)
        zc = jnp.zeros((1, HID), jnp.float32)
        zd = jnp.zeros((NBIN, HID), jnp.float32)
        deltap = jnp.concatenate(
            [jnp.concatenate([delta if i == k else zd for k in range(PK)],
                             axis=1) for i in range(PK)], axis=0)
        constp = jnp.concatenate([const] * PK, axis=1)
        bin_p = constp + jnp.dot(bitsp, deltap,
                                 preferred_element_type=jnp.float32)

        off = 2 * NBIN
        t_spc = lax.dot_general(espc_ref[...], w1[:, off:off + DBIG], _dn,
                                preferred_element_type=jnp.float32)
        t_nta = lax.dot_general(enta_ref[...], w1[:, off + DBIG:off + 2 * DBIG],
                                _dn, preferred_element_type=jnp.float32)

        def pack_rows(t, vp):
            v = t.shape[0]
            cols = []
            for a in range(PK):
                ji = lax.broadcasted_iota(jnp.int32, (vp // PK, v), 0)
                ri = lax.broadcasted_iota(jnp.int32, (vp // PK, v), 1)
                sel = (ri == PK * ji + a).astype(jnp.float32)
                cols.append(jnp.dot(sel, t, preferred_element_type=jnp.float32))
            return jnp.concatenate(cols, axis=1)

        spc_p = pack_rows(t_spc, VSPC4)
        nta_p = pack_rows(t_nta, VNTA)
        wnum = w1[:, TOT - 1:TOT]
        wnum_row = lax.dot_general(jnp.ones((1, 1), jnp.float32), wnum, _dn,
                                   preferred_element_type=jnp.float32)
        wnum_p = jnp.concatenate([wnum_row] * PK, axis=1)
        tab_ref[...] = jnp.concatenate([bin_p, spc_p, nta_p, wnum_p], axis=0)
        zc = jnp.zeros((1, HID), jnp.float32)
        mn_ref[...] = jnp.concatenate(
            [jnp.concatenate([wnum_row if i == k else zc for k in range(PK)],
                             axis=1) for i in range(PK)], axis=0)

        w2 = w2_ref[...]
        z32 = jnp.zeros((HID, HID), jnp.float32)
        w2p_ref[...] = jnp.concatenate(
            [jnp.concatenate([w2 if i == k else z32 for k in range(PK)], axis=1)
             for i in range(PK)], axis=0)
        w3 = w3_ref[...]
        z1 = jnp.zeros((1, HID), jnp.float32)
        w3p_ref[...] = jnp.concatenate(
            [jnp.concatenate([w3[o:o + 1] if k == a else z1
                              for k in range(PK)], axis=1)
             for o in range(OUT) for a in range(PK)], axis=0)
        b2r = b2_ref[...][None, :]
        b2p_ref[...] = jnp.concatenate([b2r] * PK, axis=1)
        b3r = b3_ref[...][None, :]
        b3p_ref[...] = jnp.concatenate(
            [b3r[:, o:o + 1] for o in range(OUT) for a in range(PK)],
            axis=1)

    out_shapes = (
        jax.ShapeDtypeStruct((VTOT // PK + 1, PK * HID), jnp.float32),
        jax.ShapeDtypeStruct((PK * HID, PK * HID), jnp.float32),
        jax.ShapeDtypeStruct((PK * OUT, PK * HID), jnp.float32),
        jax.ShapeDtypeStruct((PK, PK * HID), jnp.float32),
        jax.ShapeDtypeStruct((1, PK * HID), jnp.float32),
        jax.ShapeDtypeStruct((1, PK * OUT), jnp.float32),
    )
    return pl.pallas_call(body, out_shape=out_shapes)(
        *bin_embs, emb_spc, emb_nta, W1, b1, W2, b2, W3, b3)


def _sc_gather_sum(idxs_and_table):
    mesh = plsc.VectorSubcoreMesh(core_axis_name="c", subcore_axis_name="s")

    scratch = [pltpu.VMEM((BPW,), jnp.int32) for _ in range(NBIN + 2)]
    scratch += [pltpu.VMEM((BPW,), jnp.int32) for _ in range(3)]
    scratch += [pltpu.VMEM((BPW, HID), jnp.float32) for _ in range(3)]
    scratch.append(pltpu.VMEM((PPW, PK * HID), jnp.float32))
    scratch += [pltpu.SemaphoreType.DMA for _ in range(NG)]
    scratch += [pltpu.SemaphoreType.DMA for _ in range(NG)]
    scratch.append(pltpu.SemaphoreType.DMA)

    @functools.partial(
        pl.kernel,
        out_type=jax.ShapeDtypeStruct((B4, PK * HID), jnp.float32),
        mesh=mesh,
        scratch_types=scratch,
        compiler_params=pltpu.CompilerParams(use_tc_tiling_on_sc=False),
    )
    def body(*refs):
        idx_hbm = refs[:NBIN + 2]
        tab_hbm = refs[NBIN + 2]
        out = refs[NBIN + 3]
        idx_v = refs[NBIN + 4:2 * NBIN + 6]
        key_v = refs[2 * NBIN + 6:2 * NBIN + 9]
        r = refs[2 * NBIN + 9:2 * NBIN + 12]
        rp = refs[2 * NBIN + 12]
        isems = refs[2 * NBIN + 13:2 * NBIN + 13 + NG]
        gsems = refs[2 * NBIN + 13 + NG:2 * NBIN + 13 + 2 * NG]
        osem = refs[2 * NBIN + 13 + 2 * NG]

        wid = lax.axis_index("s") * NC + lax.axis_index("c")
        order = [NBIN, NBIN + 1] + list(range(NBIN))
        idx_cps = {}
        for c in order:
            idx_cps[c] = [
                pltpu.async_copy(idx_hbm[c].at[pl.ds(B4 * a + GR * wid, GR)],
                                 idx_v[c].at[pl.ds(a * GR, GR)], isems[a])
                for a in range(NG)]
        copies = [[None] * 3 for _ in range(NG)]
        for g in range(NG):
            for c in order:
                idx_cps[c][g].wait()

            def keys(k, _):
                lanes = pl.ds(k * L, L)
                key_v[1][lanes] = idx_v[NBIN][lanes] + OFF_SPC
                key_v[2][lanes] = idx_v[NBIN + 1][lanes] + OFF_NTA
                acc = idx_v[0][lanes]
                for c in range(1, NBIN):
                    acc = acc | (idx_v[c][lanes] << c)
                key_v[0][lanes] = acc
                return 0
            lax.fori_loop(g * (GR // L), (g + 1) * (GR // L), keys, 0,
                          unroll=2)
            rows = pl.ds(g * GR, GR)
            for t in range(3):
                copies[g][t] = pltpu.async_copy(
                    tab_hbm.at[key_v[t].at[rows]], r[t].at[rows], gsems[g])

        for g in range(NG):
            for t in range(3):
                copies[g][t].wait()

            @plsc.parallel_loop(0, PPW, 1, unroll=2)
            def _pack(pj, g=g):
                i = g * GR + pj
                for half in range(HID // L):
                    s = pl.ds(half * L, L)
                    rp[pj, pl.ds(g * HID + half * L, L)] = (
                        r[0][i, s] + r[1][i, s] + r[2][i, s])

        pltpu.async_copy(rp, out.at[pl.ds(wid * PPW, PPW)], osem).wait()

    return body(*idxs_and_table)


def _tc_mlp(h4, n128, w2p, b2p, w3p, b3p, mn):
    BR4 = B4
    NB = BR4 // 128

    def body(h_ref, n0_ref, n1_ref, n2_ref, n3_ref, w2_ref, b2_ref,
             w3_ref, b3_ref, mn_ref, o_ref):
        m1a = lax.broadcasted_iota(jnp.int32, (BR4, NB), 0) // 128
        m1b = lax.broadcasted_iota(jnp.int32, (BR4, NB), 1)
        m1 = (m1a == m1b).astype(jnp.float32)
        da = lax.broadcasted_iota(jnp.int32, (BR4, 128), 0) % 128
        db = lax.broadcasted_iota(jnp.int32, (BR4, 128), 1)
        dmask = (da == db).astype(jnp.float32)
        cols = []
        for n_ref in (n0_ref, n1_ref, n2_ref, n3_ref):
            spread = jnp.dot(m1, n_ref[...],
                             preferred_element_type=jnp.float32)
            cols.append(jnp.sum(spread * dmask, axis=1, keepdims=True))
        numt4 = jnp.concatenate(cols, axis=1)

        h = h_ref[...] + jnp.dot(numt4, mn_ref[...],
                                 preferred_element_type=jnp.float32)
        h = jnp.maximum(h, 0.0)
        h = lax.dot_general(h, w2_ref[...], _dn,
                            preferred_element_type=jnp.float32) + b2_ref[...]
        h = jnp.maximum(h, 0.0)
        lo = lax.dot_general(h, w3_ref[...], _dn,
                             preferred_element_type=jnp.float32) + b3_ref[...]
        m = jnp.max(lo, axis=1, keepdims=True)
        e = jnp.exp(lo - m)
        qa = lax.broadcasted_iota(jnp.int32, (PK * OUT, PK * OUT), 0) % PK
        qb = lax.broadcasted_iota(jnp.int32, (PK * OUT, PK * OUT), 1) % PK
        q = (qa == qb).astype(jnp.float32)
        den = jnp.dot(e, q, preferred_element_type=jnp.float32)
        o_ref[...] = jnp.transpose(e / den)

    rep = lambda shape: pl.BlockSpec(shape, lambda i: tuple(0 for _ in shape))
    nspec = lambda a: pl.BlockSpec((NB, 128), lambda i, a=a: (a, 0))
    return pl.pallas_call(
        body,
        grid=(B4 // BR4,),
        in_specs=[
            pl.BlockSpec((BR4, PK * HID), lambda i: (i, 0)),
            nspec(0), nspec(1), nspec(2), nspec(3),
            rep((PK * HID, PK * HID)),
            rep((1, PK * HID)),
            rep((PK * OUT, PK * HID)),
            rep((1, PK * OUT)),
            rep((PK, PK * HID)),
        ],
        out_specs=pl.BlockSpec((PK * OUT, BR4), lambda i: (0, i)),
        out_shape=jax.ShapeDtypeStruct((PK * OUT, B4), jnp.float32),
    )(h4, n128, n128, n128, n128, w2p, b2p, w3p, b3p, mn)


def kernel(numerical_features,
           idx_root_stone, emb_root_stone,
           idx_root_grate, emb_root_grate,
           idx_root_other, emb_root_other,
           idx_trunk_wire, emb_trunk_wire,
           idx_trnk_light, emb_trnk_light,
           idx_trnk_other, emb_trnk_other,
           idx_brch_light, emb_brch_light,
           idx_brch_shoe, emb_brch_shoe,
           idx_brch_other, emb_brch_other,
           idx_curb_loc, emb_curb_loc,
           idx_sidewalk, emb_sidewalk,
           idx_spc_common, emb_spc_common,
           idx_nta, emb_nta,
           W1, b1, W2, b2, W3, b3):
    idxs = [idx_root_stone, idx_root_grate, idx_root_other, idx_trunk_wire,
            idx_trnk_light, idx_trnk_other, idx_brch_light, idx_brch_shoe,
            idx_brch_other, idx_curb_loc, idx_sidewalk, idx_spc_common, idx_nta]
    bin_embs = [emb_root_stone, emb_root_grate, emb_root_other, emb_trunk_wire,
                emb_trnk_light, emb_trnk_other, emb_brch_light, emb_brch_shoe,
                emb_brch_other, emb_curb_loc, emb_sidewalk]

    tab4, w2p, w3p, mn, b2p, b3p = _prep_tables(
        bin_embs, emb_spc_common, emb_nta, W1, b1, W2, b2, W3, b3)
    tab = jnp.reshape(tab4, (VTOT + PK, HID))
    idxs32 = [i.astype(jnp.int32) for i in idxs]
    h4 = _sc_gather_sum(idxs32 + [tab])
    n128 = jnp.reshape(numerical_features, (128, 128))
    o12 = _tc_mlp(h4, n128, w2p, b2p, w3p, b3p, mn)
    return jnp.transpose(jnp.reshape(o12, (OUT, B)))
